# baseline (speedup 1.0000x reference)
import jax
import jax.numpy as jnp
from jax.experimental import pallas as pl
from jax.experimental.pallas import tpu as pltpu

_BB = 16


def _fused(h_ref, m_ref, wca_ref, wf_ref, bf_ref, out_ref, haug_ref, mha_ref):
    n = m_ref.shape[2]
    d = h_ref.shape[3]
    bn = _BB * n

    @pl.when(pl.program_id(0) == 0)
    def _init():
        haug_ref[:, d:2 * d] = jnp.zeros((bn, d), jnp.bfloat16)
        haug_ref[:, d:d + 1] = jnp.ones((bn, 1), jnp.bfloat16)

    for bb in range(_BB):
        haug_ref[bb * n:(bb + 1) * n, 0:d] = h_ref[0, bb].astype(jnp.bfloat16)
    for bb in range(_BB):
        mb = m_ref[0, bb].astype(jnp.bfloat16)
        mh = jnp.dot(mb, haug_ref[bb * n:(bb + 1) * n, :],
                     preferred_element_type=jnp.float32)
        deg = mh[:, d:d + 1]
        inv = 1.0 / jnp.maximum(deg, 1.0)
        mha_ref[bb * n:(bb + 1) * n, :] = (mh * inv).astype(jnp.bfloat16)
    s = jnp.dot(mha_ref[...], wca_ref[...], preferred_element_type=jnp.float32)
    a = jnp.maximum(s.astype(jnp.bfloat16), 0)
    y = jnp.dot(a, wf_ref[...], preferred_element_type=jnp.float32)
    out_ref[0] = y + bf_ref[...]


def kernel(h, mask, W_conv, b_conv, W_ff, b_ff):
    B, N, D = h.shape
    H = W_conv.shape[1]
    G = B // _BB
    h2 = h.reshape(G, _BB, N, D)
    m2 = mask.reshape(G, _BB, N, N)
    wca = jnp.concatenate(
        [W_conv, b_conv[None, :], jnp.zeros((D - 1, H), W_conv.dtype)],
        axis=0).astype(jnp.bfloat16)
    out = pl.pallas_call(
        _fused,
        grid=(G,),
        in_specs=[
            pl.BlockSpec((1, _BB, N, D), lambda b: (b, 0, 0, 0)),
            pl.BlockSpec((1, _BB, N, N), lambda b: (b, 0, 0, 0)),
            pl.BlockSpec((2 * D, H), lambda b: (0, 0)),
            pl.BlockSpec((H, D), lambda b: (0, 0)),
            pl.BlockSpec((1, D), lambda b: (0, 0)),
        ],
        out_specs=pl.BlockSpec((1, _BB * N, D), lambda b: (b, 0, 0)),
        out_shape=jax.ShapeDtypeStruct((G, _BB * N, D), jnp.float32),
        scratch_shapes=[
            pltpu.VMEM((_BB * N, 2 * D), jnp.bfloat16),
            pltpu.VMEM((_BB * N, 2 * D), jnp.bfloat16),
        ],
    )(h2, m2, wca, W_ff.astype(jnp.bfloat16), b_ff.reshape(1, D))
    return out.reshape(B, N, D)


# speedup vs baseline: 1.0752x; 1.0752x over previous
import jax
import jax.numpy as jnp
from jax.experimental import pallas as pl
from jax.experimental.pallas import tpu as pltpu

_BB = 16


def _fused(h_ref, m_ref, wca_ref, wf_ref, bf_ref, out_ref, haug_ref, mha_ref):
    n = m_ref.shape[2]
    d = h_ref.shape[3]
    bn = _BB * n

    @pl.when(pl.program_id(0) == 0)
    def _init():
        haug_ref[:, d:2 * d] = jnp.zeros((bn, d), jnp.float32)
        haug_ref[:, d:d + 1] = jnp.ones((bn, 1), jnp.float32)

    for bb in range(_BB):
        haug_ref[bb * n:(bb + 1) * n, 0:d] = h_ref[0, bb]
        mh = jnp.dot(m_ref[0, bb], haug_ref[bb * n:(bb + 1) * n, :],
                     preferred_element_type=jnp.float32)
        deg = mh[:, d:d + 1]
        inv = 1.0 / jnp.maximum(deg, 1.0)
        mha_ref[bb * n:(bb + 1) * n, :] = mh * inv
    s = jnp.dot(mha_ref[...], wca_ref[...], preferred_element_type=jnp.float32)
    a = jnp.maximum(s, 0.0)
    y = jnp.dot(a, wf_ref[...], preferred_element_type=jnp.float32)
    out_ref[0] = y + bf_ref[...]


def kernel(h, mask, W_conv, b_conv, W_ff, b_ff):
    B, N, D = h.shape
    H = W_conv.shape[1]
    G = B // _BB
    h2 = h.reshape(G, _BB, N, D)
    m2 = mask.reshape(G, _BB, N, N)
    wca = jnp.concatenate(
        [W_conv, b_conv[None, :], jnp.zeros((D - 1, H), W_conv.dtype)],
        axis=0)
    out = pl.pallas_call(
        _fused,
        grid=(G,),
        in_specs=[
            pl.BlockSpec((1, _BB, N, D), lambda b: (b, 0, 0, 0)),
            pl.BlockSpec((1, _BB, N, N), lambda b: (b, 0, 0, 0)),
            pl.BlockSpec((2 * D, H), lambda b: (0, 0)),
            pl.BlockSpec((H, D), lambda b: (0, 0)),
            pl.BlockSpec((1, D), lambda b: (0, 0)),
        ],
        out_specs=pl.BlockSpec((1, _BB * N, D), lambda b: (b, 0, 0)),
        out_shape=jax.ShapeDtypeStruct((G, _BB * N, D), jnp.float32),
        scratch_shapes=[
            pltpu.VMEM((_BB * N, 2 * D), jnp.float32),
            pltpu.VMEM((_BB * N, 2 * D), jnp.float32),
        ],
    )(h2, m2, wca, W_ff, b_ff.reshape(1, D))
    return out.reshape(B, N, D)
